# baseline (speedup 1.0000x reference)
import functools
import math

import jax
import jax.numpy as jnp
from jax import lax
from jax.experimental import pallas as pl
from jax.experimental.pallas import tpu as pltpu
from jax.experimental.pallas import tpu_sc as plsc

T, D, OUT, E, INNER = 2048, 768, 768, 8, 3072
BT = 256
G = (2 * T) // BT + E
NPAD = G * BT
NW = 32


def _gate_body(g_ref, wg_ref, bg_ref, pos_ref, w_ref, bexp_ref):
    logits = lax.dot_general(
        g_ref[...], wg_ref[...],
        dimension_numbers=(((1,), (1,)), ((), ())),
        preferred_element_type=jnp.float32,
    ) + bg_ref[...]
    m = jnp.max(logits, axis=1, keepdims=True)
    ex = jnp.exp(logits - m)
    gs = ex / jnp.sum(ex, axis=1, keepdims=True)

    col = lax.broadcasted_iota(jnp.int32, gs.shape, 1)
    m0 = jnp.max(gs, axis=1, keepdims=True)
    idx0 = jnp.min(jnp.where(gs == m0, col, E), axis=1, keepdims=True)
    gs_wo = jnp.where(col == idx0, -1.0, gs)
    m1 = jnp.max(gs_wo, axis=1, keepdims=True)
    idx1 = jnp.min(jnp.where(gs_wo == m1, col, E), axis=1, keepdims=True)

    mask = ((col == idx0) | (col == idx1)).astype(jnp.float32)
    denom = jnp.sum(gs * mask, axis=0, keepdims=True) + 1e-6
    cap = jnp.float32(T)

    def rowsel(a):
        return (jnp.sum(jnp.where(col == idx0, a, 0.0), axis=1, keepdims=True),
                jnp.sum(jnp.where(col == idx1, a, 0.0), axis=1, keepdims=True))

    d0, d1 = rowsel(jnp.broadcast_to(denom, gs.shape))
    w0 = m0 / d0 * cap
    w1 = m1 / d1 * cap

    row_t = lax.broadcasted_iota(jnp.int32, (T, T), 0)
    col_t = lax.broadcasted_iota(jnp.int32, (T, T), 1)
    lt = (row_t >= col_t).astype(jnp.bfloat16)
    oh0 = (col == idx0).astype(jnp.bfloat16)
    oh1 = (col == idx1).astype(jnp.bfloat16)
    cum0 = lax.dot_general(lt, oh0, dimension_numbers=(((1,), (0,)), ((), ())),
                           preferred_element_type=jnp.float32)
    cum1 = lax.dot_general(lt, oh1, dimension_numbers=(((1,), (0,)), ((), ())),
                           preferred_element_type=jnp.float32)
    tot0 = cum0[T - 1:T, :]
    tot1 = cum1[T - 1:T, :]
    counts = tot0 + tot1
    pc = jnp.ceil(counts / BT) * BT
    r8 = lax.broadcasted_iota(jnp.int32, (E, E), 0)
    c8 = lax.broadcasted_iota(jnp.int32, (E, E), 1)
    sut = (r8 < c8).astype(jnp.float32)
    off = lax.dot_general(pc, sut, dimension_numbers=(((1,), (0,)), ((), ())),
                          preferred_element_type=jnp.float32)

    off_b = jnp.broadcast_to(off, gs.shape)
    o0, o1 = rowsel(off_b)
    t0sel = rowsel(jnp.broadcast_to(tot0, gs.shape))[1]
    c0sel = rowsel(cum0)[0]
    c1sel = rowsel(cum1)[1]
    pos0 = o0 + c0sel - 1.0
    pos1 = o1 + t0sel + c1sel - 1.0

    pos_ref[...] = jnp.concatenate(
        [pos0.astype(jnp.int32), pos1.astype(jnp.int32)], axis=0)
    w_ref[...] = jnp.concatenate([w0, w1], axis=0)

    gbt = (lax.broadcasted_iota(jnp.int32, (G, E), 0) * BT).astype(jnp.float32)
    ind = (jnp.broadcast_to(off, (G, E)) <= gbt).astype(jnp.float32)
    bexp_ref[...] = (jnp.sum(ind, axis=1, keepdims=True) - 1.0).astype(jnp.int32)


def _gate_call(gate_input, Wg, bg):
    return pl.pallas_call(
        _gate_body,
        out_shape=[
            jax.ShapeDtypeStruct((2 * T, 1), jnp.int32),
            jax.ShapeDtypeStruct((2 * T, 1), jnp.float32),
            jax.ShapeDtypeStruct((G, 1), jnp.int32),
        ],
    )(gate_input, Wg, bg.reshape(1, E))


def _cast_body(a_ref, b_ref, ao_ref, bo_ref):
    ao_ref[...] = a_ref[...].astype(jnp.bfloat16)
    bo_ref[...] = b_ref[...].astype(jnp.bfloat16)


def _cast_weights(W1, W2):
    w1r = W1.reshape(2 * E, INNER // 2, D)
    w2r = W2.reshape(2 * E, OUT // 2, INNER)
    o1, o2 = pl.pallas_call(
        _cast_body,
        grid=(2 * E,),
        in_specs=[
            pl.BlockSpec((1, INNER // 2, D), lambda i: (i, 0, 0)),
            pl.BlockSpec((1, OUT // 2, INNER), lambda i: (i, 0, 0)),
        ],
        out_specs=[
            pl.BlockSpec((1, INNER // 2, D), lambda i: (i, 0, 0)),
            pl.BlockSpec((1, OUT // 2, INNER), lambda i: (i, 0, 0)),
        ],
        out_shape=[
            jax.ShapeDtypeStruct((2 * E, INNER // 2, D), jnp.bfloat16),
            jax.ShapeDtypeStruct((2 * E, OUT // 2, INNER), jnp.bfloat16),
        ],
    )(w1r, w2r)
    return o1.reshape(E, INNER, D), o2.reshape(E, OUT, INNER)


_R = NPAD // NW
_DW = D


def _build_gather_body(x_hbm, pos_hbm, w_hbm, ramp_hbm, zero_hbm,
                       xs_hbm, ws_hbm, pos_v, w_v, perm_v, ws_v, rows_v, sem,
                       wsem):
    nc = 2
    wid = lax.axis_index("s") * nc + lax.axis_index("c")
    pltpu.sync_copy(pos_hbm, pos_v)
    pltpu.sync_copy(w_hbm, w_v)
    pltpu.sync_copy(ramp_hbm, perm_v)
    pltpu.sync_copy(zero_hbm, ws_v)

    iota16 = lax.broadcasted_iota(jnp.int32, (16,), 0)

    def scat(k, _):
        sl = pl.ds(k * 16, 16)
        pv = pos_v[sl]
        tv = (iota16 + k * 16) & (T - 1)
        wv = w_v[sl]
        plsc.store_scatter(perm_v, [pv], tv)
        plsc.store_scatter(ws_v, [pv], wv)
        return 0

    lax.fori_loop(0, (2 * T) // 16, scat, 0)

    @pl.when(wid == 0)
    def _():
        pltpu.sync_copy(ws_v, ws_hbm)

    nch = 3
    ch = _R // nch
    bufs = rows_v
    wh = [None, None]
    for c in range(nch):
        b = c % 2
        if wh[b] is not None:
            wh[b].wait()
        base = wid * _R + c * ch
        idx_sl = perm_v.at[pl.ds(base, ch)]
        pltpu.async_copy(x_hbm.at[idx_sl], bufs[b], sem).wait()
        wh[b] = pltpu.async_copy(bufs[b], xs_hbm.at[pl.ds(base, ch)], wsem[b])
    wh[0].wait()
    wh[1].wait()


def _build_gather(x_i32, pos_flat, w_flat):
    mesh = plsc.VectorSubcoreMesh(core_axis_name="c", subcore_axis_name="s")
    ramp = jnp.arange(NPAD, dtype=jnp.int32) & (T - 1)
    zero = jnp.zeros((NPAD,), jnp.float32)
    f = functools.partial(
        pl.kernel,
        mesh=mesh,
        out_type=[
            jax.ShapeDtypeStruct((NPAD, _DW), jnp.float32),
            jax.ShapeDtypeStruct((NPAD,), jnp.float32),
        ],
        scratch_types=[
            pltpu.VMEM((2 * T,), jnp.int32),
            pltpu.VMEM((2 * T,), jnp.float32),
            pltpu.VMEM((NPAD,), jnp.int32),
            pltpu.VMEM((NPAD,), jnp.float32),
            [pltpu.VMEM((_R // 3, _DW), jnp.float32),
             pltpu.VMEM((_R // 3, _DW), jnp.float32)],
            pltpu.SemaphoreType.DMA,
            [pltpu.SemaphoreType.DMA, pltpu.SemaphoreType.DMA],
        ],
        compiler_params=pltpu.CompilerParams(needs_layout_passes=False),
    )(_build_gather_body)
    return f(x_i32, pos_flat, w_flat, ramp, zero)


_NIC = 2
_IC = INNER // _NIC


def _ffn_body(bexp_ref, xs_ref, ws_ref, w1_ref, b1_ref, w2_ref, b2_ref, o_ref,
              w1b_ref, w2b_ref):
    g = pl.program_id(0)
    ic = pl.program_id(1)
    gm1 = jnp.maximum(g - 1, 0)
    changed = (g == 0) | (bexp_ref[g] != bexp_ref[gm1])

    @pl.when(changed)
    def _():
        w1b_ref[ic] = w1_ref[0].astype(jnp.bfloat16)
        w2b_ref[ic] = w2_ref[0].astype(jnp.bfloat16)

    xb = xs_ref[...].astype(jnp.bfloat16)
    h = lax.dot_general(
        xb, w1b_ref[ic],
        dimension_numbers=(((1,), (1,)), ((), ())),
        preferred_element_type=jnp.float32,
    ) + b1_ref[0]
    h = 0.5 * h * (1.0 + lax.erf(h * (1.0 / math.sqrt(2.0))))
    y = lax.dot_general(
        h.astype(jnp.bfloat16), w2b_ref[ic],
        dimension_numbers=(((1,), (1,)), ((), ())),
        preferred_element_type=jnp.float32,
    )
    y = y + jnp.where(ic == 0, 1.0, 0.0) * b2_ref[0]
    contrib = ws_ref[...] * y

    @pl.when(ic == 0)
    def _():
        o_ref[...] = contrib

    @pl.when(ic != 0)
    def _():
        o_ref[...] = o_ref[...] + contrib


def _ffn_call(bexp, xs, ws, w1, b1r, w2, b2r):
    grid_spec = pltpu.PrefetchScalarGridSpec(
        num_scalar_prefetch=1,
        grid=(G, _NIC),
        in_specs=[
            pl.BlockSpec((BT, D), lambda g, ic, b: (g, 0)),
            pl.BlockSpec((BT, 1), lambda g, ic, b: (g, 0)),
            pl.BlockSpec((1, _IC, D), lambda g, ic, b: (b[g], ic, 0)),
            pl.BlockSpec((1, 1, _IC), lambda g, ic, b: (b[g], 0, ic)),
            pl.BlockSpec((1, OUT, _IC), lambda g, ic, b: (b[g], 0, ic)),
            pl.BlockSpec((1, 1, OUT), lambda g, ic, b: (b[g], 0, 0)),
        ],
        out_specs=pl.BlockSpec((BT, OUT), lambda g, ic, b: (g, 0)),
        scratch_shapes=[
            pltpu.VMEM((_NIC, _IC, D), jnp.bfloat16),
            pltpu.VMEM((_NIC, OUT, _IC), jnp.bfloat16),
        ],
    )
    return pl.pallas_call(
        _ffn_body,
        grid_spec=grid_spec,
        out_shape=jax.ShapeDtypeStruct((NPAD, OUT), jnp.float32),
    )(bexp, xs, ws, w1, b1r, w2, b2r)


_TPT = T // NW
_CC = _TPT // 2


def _combine_body(ys_hbm, p_hbm, out_hbm, i0_v, i1_v, r0_v, r1_v, sem):
    nc = 2
    wid = lax.axis_index("s") * nc + lax.axis_index("c")
    for c in range(_TPT // _CC):
        base = wid * _TPT + c * _CC
        pltpu.sync_copy(p_hbm.at[pl.ds(base, _CC)], i0_v)
        pltpu.sync_copy(p_hbm.at[pl.ds(T + base, _CC)], i1_v)
        pltpu.async_copy(ys_hbm.at[i0_v], r0_v, sem).wait()
        pltpu.async_copy(ys_hbm.at[i1_v], r1_v, sem).wait()

        def addrow(r, _):
            for cc in range(OUT // 16):
                sl = pl.ds(cc * 16, 16)
                r0_v[r, sl] = r0_v[r, sl] + r1_v[r, sl]
            return 0

        lax.fori_loop(0, _CC, addrow, 0)
        pltpu.sync_copy(r0_v, out_hbm.at[pl.ds(base, _CC)])


def _combine(ys, pos01):
    mesh = plsc.VectorSubcoreMesh(core_axis_name="c", subcore_axis_name="s")
    f = functools.partial(
        pl.kernel,
        mesh=mesh,
        out_type=jax.ShapeDtypeStruct((T, OUT), jnp.float32),
        scratch_types=[
            pltpu.VMEM((_CC,), jnp.int32),
            pltpu.VMEM((_CC,), jnp.int32),
            pltpu.VMEM((_CC, OUT), jnp.float32),
            pltpu.VMEM((_CC, OUT), jnp.float32),
            pltpu.SemaphoreType.DMA,
        ],
    )(_combine_body)
    return f(ys, pos01)


def kernel(x, gate_input, Wg, bg, W1, b1, W2, b2):
    pos01, w01, bexp = _gate_call(gate_input, Wg, bg)
    pos_flat = pos01.reshape(2 * T)
    w_flat = w01.reshape(2 * T)

    xs, ws = _build_gather(x, pos_flat, w_flat)

    ys = _ffn_call(
        bexp.reshape(G), xs, ws.reshape(NPAD, 1),
        W1, b1.reshape(E, 1, INNER),
        W2, b2.reshape(E, 1, OUT),
    )

    return _combine(ys, pos_flat)


# speedup vs baseline: 1.1887x; 1.1887x over previous
import functools
import math

import jax
import jax.numpy as jnp
from jax import lax
from jax.experimental import pallas as pl
from jax.experimental.pallas import tpu as pltpu
from jax.experimental.pallas import tpu_sc as plsc

T, D, OUT, E, INNER = 2048, 768, 768, 8, 3072
BT = 256
G = (2 * T) // BT + E
NPAD = G * BT
NW = 32


def _gate_body(g_ref, wg_ref, bg_ref, pos_ref, w_ref, bexp_ref):
    logits = lax.dot_general(
        g_ref[...], wg_ref[...],
        dimension_numbers=(((1,), (1,)), ((), ())),
        preferred_element_type=jnp.float32,
    ) + bg_ref[...]
    m = jnp.max(logits, axis=1, keepdims=True)
    ex = jnp.exp(logits - m)
    gs = ex / jnp.sum(ex, axis=1, keepdims=True)

    col = lax.broadcasted_iota(jnp.int32, gs.shape, 1)
    m0 = jnp.max(gs, axis=1, keepdims=True)
    idx0 = jnp.min(jnp.where(gs == m0, col, E), axis=1, keepdims=True)
    gs_wo = jnp.where(col == idx0, -1.0, gs)
    m1 = jnp.max(gs_wo, axis=1, keepdims=True)
    idx1 = jnp.min(jnp.where(gs_wo == m1, col, E), axis=1, keepdims=True)

    mask = ((col == idx0) | (col == idx1)).astype(jnp.float32)
    denom = jnp.sum(gs * mask, axis=0, keepdims=True) + 1e-6
    cap = jnp.float32(T)

    def rowsel(a):
        return (jnp.sum(jnp.where(col == idx0, a, 0.0), axis=1, keepdims=True),
                jnp.sum(jnp.where(col == idx1, a, 0.0), axis=1, keepdims=True))

    d0, d1 = rowsel(jnp.broadcast_to(denom, gs.shape))
    w0 = m0 / d0 * cap
    w1 = m1 / d1 * cap

    row_t = lax.broadcasted_iota(jnp.int32, (T, T), 0)
    col_t = lax.broadcasted_iota(jnp.int32, (T, T), 1)
    lt = (row_t >= col_t).astype(jnp.bfloat16)
    oh0 = (col == idx0).astype(jnp.bfloat16)
    oh1 = (col == idx1).astype(jnp.bfloat16)
    cum0 = lax.dot_general(lt, oh0, dimension_numbers=(((1,), (0,)), ((), ())),
                           preferred_element_type=jnp.float32)
    cum1 = lax.dot_general(lt, oh1, dimension_numbers=(((1,), (0,)), ((), ())),
                           preferred_element_type=jnp.float32)
    tot0 = cum0[T - 1:T, :]
    tot1 = cum1[T - 1:T, :]
    counts = tot0 + tot1
    pc = jnp.ceil(counts / BT) * BT
    r8 = lax.broadcasted_iota(jnp.int32, (E, E), 0)
    c8 = lax.broadcasted_iota(jnp.int32, (E, E), 1)
    sut = (r8 < c8).astype(jnp.float32)
    off = lax.dot_general(pc, sut, dimension_numbers=(((1,), (0,)), ((), ())),
                          preferred_element_type=jnp.float32)

    off_b = jnp.broadcast_to(off, gs.shape)
    o0, o1 = rowsel(off_b)
    t0sel = rowsel(jnp.broadcast_to(tot0, gs.shape))[1]
    c0sel = rowsel(cum0)[0]
    c1sel = rowsel(cum1)[1]
    pos0 = o0 + c0sel - 1.0
    pos1 = o1 + t0sel + c1sel - 1.0

    pos_ref[...] = jnp.concatenate(
        [pos0.astype(jnp.int32), pos1.astype(jnp.int32)], axis=0)
    w_ref[...] = jnp.concatenate([w0, w1], axis=0)

    gbt = (lax.broadcasted_iota(jnp.int32, (G, E), 0) * BT).astype(jnp.float32)
    ind = (jnp.broadcast_to(off, (G, E)) <= gbt).astype(jnp.float32)
    bexp_ref[...] = (jnp.sum(ind, axis=1, keepdims=True) - 1.0).astype(jnp.int32)


def _gate_call(gate_input, Wg, bg):
    return pl.pallas_call(
        _gate_body,
        out_shape=[
            jax.ShapeDtypeStruct((2 * T, 1), jnp.int32),
            jax.ShapeDtypeStruct((2 * T, 1), jnp.float32),
            jax.ShapeDtypeStruct((G, 1), jnp.int32),
        ],
    )(gate_input, Wg, bg.reshape(1, E))


def _cast_body(a_ref, b_ref, ao_ref, bo_ref):
    ao_ref[...] = a_ref[...].astype(jnp.bfloat16)
    bo_ref[...] = b_ref[...].astype(jnp.bfloat16)


def _cast_weights(W1, W2):
    w1r = W1.reshape(2 * E, INNER // 2, D)
    w2r = W2.reshape(2 * E, OUT // 2, INNER)
    o1, o2 = pl.pallas_call(
        _cast_body,
        grid=(2 * E,),
        in_specs=[
            pl.BlockSpec((1, INNER // 2, D), lambda i: (i, 0, 0)),
            pl.BlockSpec((1, OUT // 2, INNER), lambda i: (i, 0, 0)),
        ],
        out_specs=[
            pl.BlockSpec((1, INNER // 2, D), lambda i: (i, 0, 0)),
            pl.BlockSpec((1, OUT // 2, INNER), lambda i: (i, 0, 0)),
        ],
        out_shape=[
            jax.ShapeDtypeStruct((2 * E, INNER // 2, D), jnp.bfloat16),
            jax.ShapeDtypeStruct((2 * E, OUT // 2, INNER), jnp.bfloat16),
        ],
    )(w1r, w2r)
    return o1.reshape(E, INNER, D), o2.reshape(E, OUT, INNER)


_R = NPAD // NW
_DW = D


def _build_gather_body(x_hbm, pos_hbm, w_hbm, ramp_hbm, zero_hbm,
                       xs_hbm, ws_hbm, pos_v, w_v, perm_v, ws_v, rows_v, sem,
                       wsem):
    nc = 2
    wid = lax.axis_index("s") * nc + lax.axis_index("c")
    pltpu.sync_copy(pos_hbm, pos_v)
    pltpu.sync_copy(w_hbm, w_v)
    pltpu.sync_copy(ramp_hbm, perm_v)
    pltpu.sync_copy(zero_hbm, ws_v)

    iota16 = lax.broadcasted_iota(jnp.int32, (16,), 0)

    def scat(k, _):
        sl = pl.ds(k * 16, 16)
        pv = pos_v[sl]
        tv = (iota16 + k * 16) & (T - 1)
        wv = w_v[sl]
        plsc.store_scatter(perm_v, [pv], tv)
        plsc.store_scatter(ws_v, [pv], wv)
        return 0

    lax.fori_loop(0, (2 * T) // 16, scat, 0)

    @pl.when(wid == 0)
    def _():
        pltpu.sync_copy(ws_v, ws_hbm)

    nch = 3
    ch = _R // nch
    bufs = rows_v
    wh = [None, None]
    for c in range(nch):
        b = c % 2
        if wh[b] is not None:
            wh[b].wait()
        base = wid * _R + c * ch
        idx_sl = perm_v.at[pl.ds(base, ch)]
        pltpu.async_copy(x_hbm.at[idx_sl], bufs[b], sem).wait()
        wh[b] = pltpu.async_copy(bufs[b], xs_hbm.at[pl.ds(base, ch)], wsem[b])
    wh[0].wait()
    wh[1].wait()


def _build_gather(x_i32, pos_flat, w_flat):
    mesh = plsc.VectorSubcoreMesh(core_axis_name="c", subcore_axis_name="s")
    ramp = jnp.arange(NPAD, dtype=jnp.int32) & (T - 1)
    zero = jnp.zeros((NPAD,), jnp.float32)
    f = functools.partial(
        pl.kernel,
        mesh=mesh,
        out_type=[
            jax.ShapeDtypeStruct((NPAD, _DW), jnp.float32),
            jax.ShapeDtypeStruct((NPAD,), jnp.float32),
        ],
        scratch_types=[
            pltpu.VMEM((2 * T,), jnp.int32),
            pltpu.VMEM((2 * T,), jnp.float32),
            pltpu.VMEM((NPAD,), jnp.int32),
            pltpu.VMEM((NPAD,), jnp.float32),
            [pltpu.VMEM((_R // 3, _DW), jnp.float32),
             pltpu.VMEM((_R // 3, _DW), jnp.float32)],
            pltpu.SemaphoreType.DMA,
            [pltpu.SemaphoreType.DMA, pltpu.SemaphoreType.DMA],
        ],
        compiler_params=pltpu.CompilerParams(needs_layout_passes=False),
    )(_build_gather_body)
    return f(x_i32, pos_flat, w_flat, ramp, zero)


def _ffn_body(bexp_ref, xs_ref, ws_ref, w1_ref, b1_ref, w2_ref, b2_ref, o_ref,
              w1b_ref, w2b_ref):
    g = pl.program_id(0)
    gm1 = jnp.maximum(g - 1, 0)
    changed = (g == 0) | (bexp_ref[g] != bexp_ref[gm1])

    @pl.when(changed)
    def _():
        w1b_ref[...] = w1_ref[0].astype(jnp.bfloat16)
        w2b_ref[...] = w2_ref[0].astype(jnp.bfloat16)

    xb = xs_ref[...].astype(jnp.bfloat16)
    h = lax.dot_general(
        xb, w1b_ref[...],
        dimension_numbers=(((1,), (1,)), ((), ())),
        preferred_element_type=jnp.float32,
    ) + b1_ref[0]
    h = 0.5 * h * (1.0 + lax.erf(h * (1.0 / math.sqrt(2.0))))
    y = lax.dot_general(
        h.astype(jnp.bfloat16), w2b_ref[...],
        dimension_numbers=(((1,), (1,)), ((), ())),
        preferred_element_type=jnp.float32,
    ) + b2_ref[0]
    o_ref[...] = ws_ref[...] * y


def _ffn_call(bexp, xs, ws, w1, b1r, w2, b2r):
    grid_spec = pltpu.PrefetchScalarGridSpec(
        num_scalar_prefetch=1,
        grid=(G,),
        in_specs=[
            pl.BlockSpec((BT, D), lambda g, b: (g, 0)),
            pl.BlockSpec((BT, 1), lambda g, b: (g, 0)),
            pl.BlockSpec((1, INNER, D), lambda g, b: (b[g], 0, 0)),
            pl.BlockSpec((1, 1, INNER), lambda g, b: (b[g], 0, 0)),
            pl.BlockSpec((1, OUT, INNER), lambda g, b: (b[g], 0, 0)),
            pl.BlockSpec((1, 1, OUT), lambda g, b: (b[g], 0, 0)),
        ],
        out_specs=pl.BlockSpec((BT, OUT), lambda g, b: (g, 0)),
        scratch_shapes=[
            pltpu.VMEM((INNER, D), jnp.bfloat16),
            pltpu.VMEM((OUT, INNER), jnp.bfloat16),
        ],
    )
    return pl.pallas_call(
        _ffn_body,
        grid_spec=grid_spec,
        out_shape=jax.ShapeDtypeStruct((NPAD, OUT), jnp.float32),
    )(bexp, xs, ws, w1, b1r, w2, b2r)


_TPT = T // NW
_CC = _TPT // 2


def _combine_body(ys_hbm, p_hbm, out_hbm, i0_v, i1_v, r0_v, r1_v, sem):
    nc = 2
    wid = lax.axis_index("s") * nc + lax.axis_index("c")
    for c in range(_TPT // _CC):
        base = wid * _TPT + c * _CC
        pltpu.sync_copy(p_hbm.at[pl.ds(base, _CC)], i0_v)
        pltpu.sync_copy(p_hbm.at[pl.ds(T + base, _CC)], i1_v)
        pltpu.async_copy(ys_hbm.at[i0_v], r0_v, sem).wait()
        pltpu.async_copy(ys_hbm.at[i1_v], r1_v, sem).wait()

        def addrow(r, _):
            for cc in range(OUT // 16):
                sl = pl.ds(cc * 16, 16)
                r0_v[r, sl] = r0_v[r, sl] + r1_v[r, sl]
            return 0

        lax.fori_loop(0, _CC, addrow, 0)
        pltpu.sync_copy(r0_v, out_hbm.at[pl.ds(base, _CC)])


def _combine(ys, pos01):
    mesh = plsc.VectorSubcoreMesh(core_axis_name="c", subcore_axis_name="s")
    f = functools.partial(
        pl.kernel,
        mesh=mesh,
        out_type=jax.ShapeDtypeStruct((T, OUT), jnp.float32),
        scratch_types=[
            pltpu.VMEM((_CC,), jnp.int32),
            pltpu.VMEM((_CC,), jnp.int32),
            pltpu.VMEM((_CC, OUT), jnp.float32),
            pltpu.VMEM((_CC, OUT), jnp.float32),
            pltpu.SemaphoreType.DMA,
        ],
    )(_combine_body)
    return f(ys, pos01)


def kernel(x, gate_input, Wg, bg, W1, b1, W2, b2):
    pos01, w01, bexp = _gate_call(gate_input, Wg, bg)
    pos_flat = pos01.reshape(2 * T)
    w_flat = w01.reshape(2 * T)

    xs, ws = _build_gather(x, pos_flat, w_flat)

    ys = _ffn_call(
        bexp.reshape(G), xs, ws.reshape(NPAD, 1),
        W1, b1.reshape(E, 1, INNER),
        W2, b2.reshape(E, 1, OUT),
    )

    return _combine(ys, pos_flat)


# speedup vs baseline: 1.1906x; 1.0015x over previous
import functools
import math

import jax
import jax.numpy as jnp
from jax import lax
from jax.experimental import pallas as pl
from jax.experimental.pallas import tpu as pltpu
from jax.experimental.pallas import tpu_sc as plsc

T, D, OUT, E, INNER = 2048, 768, 768, 8, 3072
BT = 256
G = (2 * T) // BT + E
NPAD = G * BT
NW = 32


def _gate_body(g_ref, wg_ref, bg_ref, pos_ref, w_ref, bexp_ref):
    logits = lax.dot_general(
        g_ref[...], wg_ref[...],
        dimension_numbers=(((1,), (1,)), ((), ())),
        preferred_element_type=jnp.float32,
    ) + bg_ref[...]
    m = jnp.max(logits, axis=1, keepdims=True)
    ex = jnp.exp(logits - m)
    gs = ex / jnp.sum(ex, axis=1, keepdims=True)

    col = lax.broadcasted_iota(jnp.int32, gs.shape, 1)
    m0 = jnp.max(gs, axis=1, keepdims=True)
    idx0 = jnp.min(jnp.where(gs == m0, col, E), axis=1, keepdims=True)
    gs_wo = jnp.where(col == idx0, -1.0, gs)
    m1 = jnp.max(gs_wo, axis=1, keepdims=True)
    idx1 = jnp.min(jnp.where(gs_wo == m1, col, E), axis=1, keepdims=True)

    mask = ((col == idx0) | (col == idx1)).astype(jnp.float32)
    denom = jnp.sum(gs * mask, axis=0, keepdims=True) + 1e-6
    cap = jnp.float32(T)

    def rowsel(a):
        return (jnp.sum(jnp.where(col == idx0, a, 0.0), axis=1, keepdims=True),
                jnp.sum(jnp.where(col == idx1, a, 0.0), axis=1, keepdims=True))

    d0, d1 = rowsel(jnp.broadcast_to(denom, gs.shape))
    w0 = m0 / d0 * cap
    w1 = m1 / d1 * cap

    row_t = lax.broadcasted_iota(jnp.int32, (T, T), 0)
    col_t = lax.broadcasted_iota(jnp.int32, (T, T), 1)
    lt = (row_t >= col_t).astype(jnp.bfloat16)
    oh0 = (col == idx0).astype(jnp.bfloat16)
    oh1 = (col == idx1).astype(jnp.bfloat16)
    cum0 = lax.dot_general(lt, oh0, dimension_numbers=(((1,), (0,)), ((), ())),
                           preferred_element_type=jnp.float32)
    cum1 = lax.dot_general(lt, oh1, dimension_numbers=(((1,), (0,)), ((), ())),
                           preferred_element_type=jnp.float32)
    tot0 = cum0[T - 1:T, :]
    tot1 = cum1[T - 1:T, :]
    counts = tot0 + tot1
    pc = jnp.ceil(counts / BT) * BT
    r8 = lax.broadcasted_iota(jnp.int32, (E, E), 0)
    c8 = lax.broadcasted_iota(jnp.int32, (E, E), 1)
    sut = (r8 < c8).astype(jnp.float32)
    off = lax.dot_general(pc, sut, dimension_numbers=(((1,), (0,)), ((), ())),
                          preferred_element_type=jnp.float32)

    off_b = jnp.broadcast_to(off, gs.shape)
    o0, o1 = rowsel(off_b)
    t0sel = rowsel(jnp.broadcast_to(tot0, gs.shape))[1]
    c0sel = rowsel(cum0)[0]
    c1sel = rowsel(cum1)[1]
    pos0 = o0 + c0sel - 1.0
    pos1 = o1 + t0sel + c1sel - 1.0

    pos_ref[...] = jnp.concatenate(
        [pos0.astype(jnp.int32), pos1.astype(jnp.int32)], axis=0)
    w_ref[...] = jnp.concatenate([w0, w1], axis=0)

    gbt = (lax.broadcasted_iota(jnp.int32, (G, E), 0) * BT).astype(jnp.float32)
    ind = (jnp.broadcast_to(off, (G, E)) <= gbt).astype(jnp.float32)
    bexp_ref[...] = (jnp.sum(ind, axis=1, keepdims=True) - 1.0).astype(jnp.int32)


def _gate_call(gate_input, Wg, bg):
    return pl.pallas_call(
        _gate_body,
        out_shape=[
            jax.ShapeDtypeStruct((2 * T, 1), jnp.int32),
            jax.ShapeDtypeStruct((2 * T, 1), jnp.float32),
            jax.ShapeDtypeStruct((G, 1), jnp.int32),
        ],
    )(gate_input, Wg, bg.reshape(1, E))


_R = NPAD // NW
_DW = D


def _build_gather_body(x_hbm, pos_hbm, w_hbm, ramp_hbm, zero_hbm,
                       xs_hbm, ws_hbm, pos_v, w_v, perm_v, ws_v, rows_v, sem,
                       wsem):
    nc = 2
    wid = lax.axis_index("s") * nc + lax.axis_index("c")
    pltpu.sync_copy(pos_hbm, pos_v)
    pltpu.sync_copy(w_hbm, w_v)
    pltpu.sync_copy(ramp_hbm, perm_v)
    pltpu.sync_copy(zero_hbm, ws_v)

    iota16 = lax.broadcasted_iota(jnp.int32, (16,), 0)

    def scat(k, _):
        sl = pl.ds(k * 16, 16)
        pv = pos_v[sl]
        tv = (iota16 + k * 16) & (T - 1)
        wv = w_v[sl]
        plsc.store_scatter(perm_v, [pv], tv)
        plsc.store_scatter(ws_v, [pv], wv)
        return 0

    lax.fori_loop(0, (2 * T) // 16, scat, 0)

    @pl.when(wid == 0)
    def _():
        pltpu.sync_copy(ws_v, ws_hbm)

    nch = 3
    ch = _R // nch
    bufs = rows_v
    wh = [None, None]
    for c in range(nch):
        b = c % 2
        if wh[b] is not None:
            wh[b].wait()
        base = wid * _R + c * ch
        idx_sl = perm_v.at[pl.ds(base, ch)]
        pltpu.async_copy(x_hbm.at[idx_sl], bufs[b], sem).wait()
        wh[b] = pltpu.async_copy(bufs[b], xs_hbm.at[pl.ds(base, ch)], wsem[b])
    wh[0].wait()
    wh[1].wait()


def _build_gather(x_i32, pos_flat, w_flat):
    mesh = plsc.VectorSubcoreMesh(core_axis_name="c", subcore_axis_name="s")
    ramp = jnp.arange(NPAD, dtype=jnp.int32) & (T - 1)
    zero = jnp.zeros((NPAD,), jnp.float32)
    f = functools.partial(
        pl.kernel,
        mesh=mesh,
        out_type=[
            jax.ShapeDtypeStruct((NPAD, _DW), jnp.float32),
            jax.ShapeDtypeStruct((NPAD,), jnp.float32),
        ],
        scratch_types=[
            pltpu.VMEM((2 * T,), jnp.int32),
            pltpu.VMEM((2 * T,), jnp.float32),
            pltpu.VMEM((NPAD,), jnp.int32),
            pltpu.VMEM((NPAD,), jnp.float32),
            [pltpu.VMEM((_R // 3, _DW), jnp.float32),
             pltpu.VMEM((_R // 3, _DW), jnp.float32)],
            pltpu.SemaphoreType.DMA,
            [pltpu.SemaphoreType.DMA, pltpu.SemaphoreType.DMA],
        ],
        compiler_params=pltpu.CompilerParams(needs_layout_passes=False),
    )(_build_gather_body)
    return f(x_i32, pos_flat, w_flat, ramp, zero)


def _ffn_body(bexp_ref, xs_ref, ws_ref, w1_ref, b1_ref, w2_ref, b2_ref, o_ref,
              w1b_ref, w2b_ref):
    g = pl.program_id(0)
    gm1 = jnp.maximum(g - 1, 0)
    changed = (g == 0) | (bexp_ref[g] != bexp_ref[gm1])

    @pl.when(changed)
    def _():
        w1b_ref[...] = w1_ref[0].astype(jnp.bfloat16)
        w2b_ref[...] = w2_ref[0].astype(jnp.bfloat16)

    xb = xs_ref[...].astype(jnp.bfloat16)
    h = lax.dot_general(
        xb, w1b_ref[...],
        dimension_numbers=(((1,), (1,)), ((), ())),
        preferred_element_type=jnp.float32,
    ) + b1_ref[0]
    h = 0.5 * h * (1.0 + lax.erf(h * (1.0 / math.sqrt(2.0))))
    y = lax.dot_general(
        h.astype(jnp.bfloat16), w2b_ref[...],
        dimension_numbers=(((1,), (1,)), ((), ())),
        preferred_element_type=jnp.float32,
    ) + b2_ref[0]
    o_ref[...] = ws_ref[...] * y


def _ffn_call(bexp, xs, ws, w1, b1r, w2, b2r):
    grid_spec = pltpu.PrefetchScalarGridSpec(
        num_scalar_prefetch=1,
        grid=(G,),
        in_specs=[
            pl.BlockSpec((BT, D), lambda g, b: (g, 0)),
            pl.BlockSpec((BT, 1), lambda g, b: (g, 0)),
            pl.BlockSpec((1, INNER, D), lambda g, b: (b[g], 0, 0)),
            pl.BlockSpec((1, 1, INNER), lambda g, b: (b[g], 0, 0)),
            pl.BlockSpec((1, OUT, INNER), lambda g, b: (b[g], 0, 0)),
            pl.BlockSpec((1, 1, OUT), lambda g, b: (b[g], 0, 0)),
        ],
        out_specs=pl.BlockSpec((BT, OUT), lambda g, b: (g, 0)),
        scratch_shapes=[
            pltpu.VMEM((INNER, D), jnp.bfloat16),
            pltpu.VMEM((OUT, INNER), jnp.bfloat16),
        ],
    )
    return pl.pallas_call(
        _ffn_body,
        grid_spec=grid_spec,
        out_shape=jax.ShapeDtypeStruct((NPAD, OUT), jnp.float32),
    )(bexp, xs, ws, w1, b1r, w2, b2r)


_TPT = T // NW
_CC = _TPT // 2


def _combine_body(ys_hbm, p_hbm, out_hbm, i0_v, i1_v, r0_v, r1_v, sem):
    nc = 2
    wid = lax.axis_index("s") * nc + lax.axis_index("c")
    for c in range(_TPT // _CC):
        base = wid * _TPT + c * _CC
        pltpu.sync_copy(p_hbm.at[pl.ds(base, _CC)], i0_v)
        pltpu.sync_copy(p_hbm.at[pl.ds(T + base, _CC)], i1_v)
        pltpu.async_copy(ys_hbm.at[i0_v], r0_v, sem).wait()
        pltpu.async_copy(ys_hbm.at[i1_v], r1_v, sem).wait()

        def addrow(r, _):
            for cc in range(OUT // 16):
                sl = pl.ds(cc * 16, 16)
                r0_v[r, sl] = r0_v[r, sl] + r1_v[r, sl]
            return 0

        lax.fori_loop(0, _CC, addrow, 0)
        pltpu.sync_copy(r0_v, out_hbm.at[pl.ds(base, _CC)])


def _combine(ys, pos01):
    mesh = plsc.VectorSubcoreMesh(core_axis_name="c", subcore_axis_name="s")
    f = functools.partial(
        pl.kernel,
        mesh=mesh,
        out_type=jax.ShapeDtypeStruct((T, OUT), jnp.float32),
        scratch_types=[
            pltpu.VMEM((_CC,), jnp.int32),
            pltpu.VMEM((_CC,), jnp.int32),
            pltpu.VMEM((_CC, OUT), jnp.float32),
            pltpu.VMEM((_CC, OUT), jnp.float32),
            pltpu.SemaphoreType.DMA,
        ],
    )(_combine_body)
    return f(ys, pos01)


def kernel(x, gate_input, Wg, bg, W1, b1, W2, b2):
    pos01, w01, bexp = _gate_call(gate_input, Wg, bg)
    pos_flat = pos01.reshape(2 * T)
    w_flat = w01.reshape(2 * T)

    xs, ws = _build_gather(x, pos_flat, w_flat)

    ys = _ffn_call(
        bexp.reshape(G), xs, ws.reshape(NPAD, 1),
        W1, b1.reshape(E, 1, INNER),
        W2, b2.reshape(E, 1, OUT),
    )

    return _combine(ys, pos_flat)
